# Initial kernel scaffold; baseline (speedup 1.0000x reference)
import jax
import jax.numpy as jnp
from jax.experimental import pallas as pl


def kernel(x, edge_index, batch, W0, b0, W1, b1, W2, b2, g0, be0, g1, be1, g2, be2, fc1_w, fc1_b, fc2_w, fc2_b):
    raise NotImplementedError("write your pallas kernel here")


# speedup vs baseline: 9.0802x; 9.0802x over previous
import functools
import jax
import jax.numpy as jnp
from jax import lax
from jax.experimental import pallas as pl
from jax.experimental.pallas import tpu as pltpu
from jax.experimental.pallas import tpu_sc as plsc

NC = 2
NS = 16
NW = NC * NS
CH = 128


def _cdiv(a, b):
    return (a + b - 1) // b


def _make_deg_kernel(n_pad, ept):
    nchunk = ept // CH
    rpt = n_pad // NS
    zc = rpt // CH
    mesh = plsc.VectorSubcoreMesh(
        core_axis_name="c", subcore_axis_name="s",
        num_cores=NC, num_subcores=NS)

    @functools.partial(
        pl.kernel,
        out_type=jax.ShapeDtypeStruct((NC, n_pad, 128), jnp.float32),
        mesh=mesh,
        scratch_types=[
            pltpu.VMEM((CH,), jnp.int32),
            pltpu.VMEM((CH, 128), jnp.float32),
            pltpu.VMEM_SHARED((n_pad, 128), jnp.float32),
        ],
    )
    def deg_kernel(dst_hbm, out_hbm, dst_v, ones_v, acc_s):
        cid = lax.axis_index("c")
        sid = lax.axis_index("s")
        wid = sid * NC + cid

        def fill(i, _):
            def lane(j, _):
                ones_v[i, pl.ds(j * 16, 16)] = jnp.zeros((16,), jnp.float32)
                return 0
            return lax.fori_loop(0, 8, lane, 0)
        lax.fori_loop(0, CH, fill, 0)

        def zinit(k, _):
            pltpu.sync_copy(ones_v, acc_s.at[pl.ds(sid * rpt + k * CH, CH)])
            return 0
        lax.fori_loop(0, zc, zinit, 0)

        def refill(i, _):
            def lane(j, _):
                ones_v[i, pl.ds(j * 16, 16)] = jnp.full((16,), 1.0,
                                                        jnp.float32)
                return 0
            return lax.fori_loop(0, 8, lane, 0)
        lax.fori_loop(0, CH, refill, 0)
        plsc.subcore_barrier()

        def body(j, _):
            pltpu.sync_copy(dst_hbm.at[pl.ds(wid * ept + j * CH, CH)], dst_v)
            pltpu.sync_copy(ones_v, acc_s.at[dst_v], add=True)
            return 0
        lax.fori_loop(0, nchunk, body, 0)
        plsc.subcore_barrier()

        pltpu.sync_copy(acc_s.at[pl.ds(sid * rpt, rpt)],
                        out_hbm.at[cid, pl.ds(sid * rpt, rpt)])

    return deg_kernel


def _make_edge_kernel(n_pad, ept, f):
    nchunk = ept // CH
    rpt = n_pad // NS
    zc = rpt // CH
    mesh = plsc.VectorSubcoreMesh(
        core_axis_name="c", subcore_axis_name="s",
        num_cores=NC, num_subcores=NS)

    @functools.partial(
        pl.kernel,
        out_type=jax.ShapeDtypeStruct((NC, n_pad, f), jnp.float32),
        mesh=mesh,
        scratch_types=[
            pltpu.VMEM((CH,), jnp.int32),
            pltpu.VMEM((CH,), jnp.int32),
            pltpu.VMEM((CH, f), jnp.float32),
            pltpu.VMEM_SHARED((n_pad, f), jnp.float32),
            pltpu.SemaphoreType.DMA,
        ],
    )
    def edge_kernel(h_hbm, src_hbm, dst_hbm, out_hbm,
                    src_v, dst_v, rows, acc_s, sem):
        cid = lax.axis_index("c")
        sid = lax.axis_index("s")
        wid = sid * NC + cid

        def zrow(i, _):
            def zlane(j, _):
                rows[i, pl.ds(j * 16, 16)] = jnp.zeros((16,), jnp.float32)
                return 0
            return lax.fori_loop(0, f // 16, zlane, 0)
        lax.fori_loop(0, CH, zrow, 0)

        def zinit(k, _):
            pltpu.sync_copy(rows, acc_s.at[pl.ds(sid * rpt + k * CH, CH)])
            return 0
        lax.fori_loop(0, zc, zinit, 0)
        plsc.subcore_barrier()

        def body(j, _):
            base = wid * ept + j * CH
            pltpu.sync_copy(src_hbm.at[pl.ds(base, CH)], src_v)
            pltpu.sync_copy(dst_hbm.at[pl.ds(base, CH)], dst_v)
            pltpu.async_copy(h_hbm.at[src_v], rows, sem).wait()
            pltpu.sync_copy(rows, acc_s.at[dst_v], add=True)
            return 0
        lax.fori_loop(0, nchunk, body, 0)
        plsc.subcore_barrier()

        pltpu.sync_copy(acc_s.at[pl.ds(sid * rpt, rpt)],
                        out_hbm.at[cid, pl.ds(sid * rpt, rpt)])

    return edge_kernel


def _head_body(n, x_ref, deg_ref, w_ref, dinv_ref, hp_ref):
    deg = deg_ref[0, :n, :] + deg_ref[1, :n, :]
    dinv = lax.rsqrt(deg[:, 0:1] + 1.0)
    dinv_ref[...] = dinv
    h = jnp.dot(x_ref[...], w_ref[...], preferred_element_type=jnp.float32)
    hp_ref[...] = h * dinv


def _post_conv(n, acc_ref, hp_ref, dinv_ref, b_ref, g_ref, be_ref):
    dinv = dinv_ref[...]
    agg = acc_ref[0, :n, :] + acc_ref[1, :n, :] + hp_ref[...]
    z = agg * dinv + b_ref[...]
    mu = jnp.mean(z, axis=0, keepdims=True)
    var = jnp.mean((z - mu) * (z - mu), axis=0, keepdims=True)
    zb = g_ref[...] * (z - mu) * lax.rsqrt(var + 1e-5) + be_ref[...]
    return jnp.maximum(zb, 0.0)


def _mid_body(n, acc_ref, hp_ref, dinv_ref, b_ref, g_ref, be_ref, wn_ref,
              out_ref):
    zr = _post_conv(n, acc_ref, hp_ref, dinv_ref, b_ref, g_ref, be_ref)
    out_ref[...] = jnp.dot(
        zr, wn_ref[...], preferred_element_type=jnp.float32) * dinv_ref[...]


def _tail_body(n, nseg, acc_ref, hp_ref, dinv_ref, b_ref, g_ref, be_ref,
               batch_ref, fc1w_ref, fc1b_ref, fc2w_ref, fc2b_ref, out_ref):
    zr = _post_conv(n, acc_ref, hp_ref, dinv_ref, b_ref, g_ref, be_ref)
    seg = lax.broadcasted_iota(jnp.int32, (nseg, n), 0)
    onehot = jnp.where(seg == batch_ref[...], 1.0, 0.0)
    sums = jnp.dot(onehot, zr, preferred_element_type=jnp.float32)
    cnt = jnp.sum(onehot, axis=1, keepdims=True)
    pooled = sums / jnp.maximum(cnt, 1.0)
    h1 = jnp.maximum(
        jnp.dot(pooled, fc1w_ref[...], preferred_element_type=jnp.float32)
        + fc1b_ref[...], 0.0)
    out_ref[...] = jnp.dot(
        h1, fc2w_ref[...], preferred_element_type=jnp.float32) + fc2b_ref[...]


def kernel(x, edge_index, batch, W0, b0, W1, b1, W2, b2,
           g0, be0, g1, be1, g2, be2, fc1_w, fc1_b, fc2_w, fc2_b):
    n, f_in = x.shape
    e = edge_index.shape[1]
    h = W0.shape[1]
    out_dim = fc2_w.shape[1]
    nseg = 16

    ept = _cdiv(e, NW * CH) * CH
    e_pad = ept * NW
    n_pad = _cdiv(n + 1, NS * CH) * NS * CH
    src = jnp.concatenate(
        [edge_index[0], jnp.zeros((e_pad - e,), jnp.int32)])
    dst = jnp.concatenate(
        [edge_index[1], jnp.full((e_pad - e,), n, jnp.int32)])

    deg2 = _make_deg_kernel(n_pad, ept)(dst)

    head = pl.pallas_call(
        functools.partial(_head_body, n),
        out_shape=[jax.ShapeDtypeStruct((n, 1), jnp.float32),
                   jax.ShapeDtypeStruct((n, h), jnp.float32)],
    )
    dinv, hp = head(x, deg2, W0)

    edge = _make_edge_kernel(n_pad, ept, h)

    mid = pl.pallas_call(
        functools.partial(_mid_body, n),
        out_shape=jax.ShapeDtypeStruct((n, h), jnp.float32),
    )
    row = lambda v: v.reshape(1, -1)
    for (b, g, be, wn) in ((b0, g0, be0, W1), (b1, g1, be1, W2)):
        acc = edge(hp, src, dst)
        hp = mid(acc, hp, dinv, row(b), row(g), row(be), wn)

    acc = edge(hp, src, dst)
    tail = pl.pallas_call(
        functools.partial(_tail_body, n, nseg),
        out_shape=jax.ShapeDtypeStruct((nseg, out_dim), jnp.float32),
    )
    return tail(acc, hp, dinv, row(b2), row(g2), row(be2),
                batch.reshape(1, -1), fc1_w, row(fc1_b), fc2_w, row(fc2_b))
